# baseline (speedup 1.0000x reference)
import functools

import jax
import jax.numpy as jnp
from jax import lax
from jax.experimental import pallas as pl
from jax.experimental.pallas import tpu as pltpu
from jax.experimental.pallas import tpu_sc as plsc

B = 128
V = 100000
C = 0.1
NWORK = B // 16

TC_V = 74912
VB = 18728
K = TC_V // VB
SC_V0 = TC_V
SC_PER_TILE = 784
SC_CHUNK = 392
NTILE = 32


def _sc_gather_body(target_hbm, predt_hbm, out_hbm, tgt_v, rows_v, diag_v, sem):
    wid = lax.axis_index("s") * 2 + lax.axis_index("c")

    @pl.when(wid < NWORK)
    def _():
        base = wid * 16
        pltpu.sync_copy(target_hbm.at[pl.ds(base, 16)], tgt_v)
        pltpu.async_copy(predt_hbm.at[tgt_v], rows_v, sem).wait()
        ii = lax.iota(jnp.int32, 16)
        d = jnp.full((16,), -C, jnp.float32)
        for l in range(16):
            vec = rows_v[l, pl.ds(base, 16)]
            d = jnp.where(ii == l, vec - C, d)
        diag_v[...] = d
        pltpu.sync_copy(diag_v, out_hbm.at[pl.ds(base, 16)])


_sc_gather = functools.partial(
    pl.kernel,
    mesh=plsc.VectorSubcoreMesh(core_axis_name="c", subcore_axis_name="s"),
    out_type=jax.ShapeDtypeStruct((B,), jnp.float32),
    scratch_types=[
        pltpu.VMEM((16,), jnp.int32),
        pltpu.VMEM((16, B), jnp.float32),
        pltpu.VMEM((16,), jnp.float32),
        pltpu.SemaphoreType.DMA,
    ],
    compiler_params=pltpu.CompilerParams(use_tc_tiling_on_sc=True),
)(_sc_gather_body)


def _sc_stream_body(corrc_hbm, predt_hbm, out_hbm,
                    cc_v, buf0_v, buf1_v, part_v, sem0, sem1):
    s = lax.axis_index("s")
    c = lax.axis_index("c")
    wid = s * 2 + c
    v0 = pl.multiple_of(SC_V0 + wid * SC_PER_TILE, 8)

    pltpu.sync_copy(corrc_hbm.at[pl.ds(0, B)], cc_v)
    ccs = [cc_v[pl.ds(16 * m, 16)] for m in range(8)]

    bufs = [buf0_v, buf1_v]
    sems = [sem0, sem1]

    def issue(k):
        return pltpu.async_copy(
            predt_hbm.at[pl.ds(v0 + k * SC_CHUNK, SC_CHUNK)],
            bufs[k], sems[k],
        )

    h0 = issue(0)
    h1 = issue(1)
    accs = [jnp.zeros((16,), jnp.float32) for _ in range(8)]
    for k in range(2):
        (h0 if k == 0 else h1).wait()
        buf = bufs[k]

        def body(i, a, buf=buf):
            base = i * 8
            out = list(a)
            for u in range(8):
                row = base + u
                for m in range(8):
                    x = buf[row, pl.ds(16 * m, 16)]
                    out[m] = out[m] + jnp.maximum(x - ccs[m], 0.0)
            return tuple(out)

        accs = list(lax.fori_loop(0, SC_CHUNK // 8, body, tuple(accs)))

    tot = accs[0]
    for m in range(1, 8):
        tot = tot + accs[m]
    part_v[...] = tot
    pltpu.sync_copy(part_v, out_hbm.at[pl.ds(wid * 16, 16)])


_sc_stream = functools.partial(
    pl.kernel,
    mesh=plsc.VectorSubcoreMesh(core_axis_name="c", subcore_axis_name="s"),
    out_type=jax.ShapeDtypeStruct((NTILE * 16,), jnp.float32),
    scratch_types=[
        pltpu.VMEM((B,), jnp.float32),
        pltpu.VMEM((SC_CHUNK, B), jnp.float32),
        pltpu.VMEM((SC_CHUNK, B), jnp.float32),
        pltpu.VMEM((16,), jnp.float32),
        pltpu.SemaphoreType.DMA,
        pltpu.SemaphoreType.DMA,
    ],
    compiler_params=pltpu.CompilerParams(use_tc_tiling_on_sc=True),
)(_sc_stream_body)


def _tc_body(corrc_ref, pred_ref, out_ref, acc_ref):
    k = pl.program_id(0)
    s = jnp.sum(jnp.maximum(pred_ref[...] - corrc_ref[...], 0.0))

    @pl.when(k == 0)
    def _():
        acc_ref[0] = s

    @pl.when(k > 0)
    def _():
        acc_ref[0] += s

    @pl.when(k == K - 1)
    def _():
        out_ref[0] = acc_ref[0]


def _final_body(tcp_ref, part_ref, out_ref):
    out_ref[0] = (tcp_ref[0] + jnp.sum(part_ref[...]) - B * C) / B


def kernel(target, prediction):
    target = target.astype(jnp.int32)
    predt = prediction.T
    corrc = _sc_gather(target, predt)
    partials = _sc_stream(corrc, predt)
    tc_part = pl.pallas_call(
        _tc_body,
        grid=(K,),
        in_specs=[
            pl.BlockSpec((1, B), lambda k: (0, 0)),
            pl.BlockSpec((VB, B), lambda k: (k, 0)),
        ],
        out_specs=pl.BlockSpec(memory_space=pltpu.SMEM),
        out_shape=jax.ShapeDtypeStruct((1,), jnp.float32),
        scratch_shapes=[pltpu.SMEM((1,), jnp.float32)],
    )(corrc.reshape(1, B), predt)
    out = pl.pallas_call(
        _final_body,
        grid=(1,),
        in_specs=[
            pl.BlockSpec(memory_space=pltpu.SMEM),
            pl.BlockSpec((NTILE, 16), lambda k: (0, 0)),
        ],
        out_specs=pl.BlockSpec(memory_space=pltpu.SMEM),
        out_shape=jax.ShapeDtypeStruct((1,), jnp.float32),
    )(tc_part, partials.reshape(NTILE, 16))
    return out


# speedup vs baseline: 1.0758x; 1.0758x over previous
import functools

import jax
import jax.numpy as jnp
from jax import lax
from jax.experimental import pallas as pl
from jax.experimental.pallas import tpu as pltpu
from jax.experimental.pallas import tpu_sc as plsc

B = 128
V = 100000
C = 0.1
VB = 25000
K = V // VB
NWORK = B // 16


def _sc_gather_body(target_hbm, predt_hbm, out_hbm, tgt_v, rows_v, diag_v, sem):
    wid = lax.axis_index("s") * 2 + lax.axis_index("c")

    @pl.when(wid < NWORK)
    def _():
        base = wid * 16
        pltpu.sync_copy(target_hbm.at[pl.ds(base, 16)], tgt_v)
        pltpu.async_copy(predt_hbm.at[tgt_v], rows_v, sem).wait()
        ii = lax.iota(jnp.int32, 16)
        d = jnp.full((16,), -C, jnp.float32)
        for l in range(16):
            vec = rows_v[l, pl.ds(base, 16)]
            d = jnp.where(ii == l, vec - C, d)
        diag_v[...] = d
        pltpu.sync_copy(diag_v, out_hbm.at[pl.ds(base, 16)])


_sc_gather = functools.partial(
    pl.kernel,
    mesh=plsc.VectorSubcoreMesh(core_axis_name="c", subcore_axis_name="s"),
    out_type=jax.ShapeDtypeStruct((B,), jnp.float32),
    scratch_types=[
        pltpu.VMEM((16,), jnp.int32),
        pltpu.VMEM((16, B), jnp.float32),
        pltpu.VMEM((16,), jnp.float32),
        pltpu.SemaphoreType.DMA,
    ],
    compiler_params=pltpu.CompilerParams(use_tc_tiling_on_sc=True),
)(_sc_gather_body)


def _tc_body(corrc_ref, pred_ref, out_ref, acc_ref):
    k = pl.program_id(0)
    s = jnp.sum(jnp.maximum(pred_ref[...] - corrc_ref[...], 0.0))

    @pl.when(k == 0)
    def _():
        acc_ref[0] = s

    @pl.when(k > 0)
    def _():
        acc_ref[0] += s

    @pl.when(k == K - 1)
    def _():
        out_ref[0] = (acc_ref[0] - B * C) / B


def kernel(target, prediction):
    target = target.astype(jnp.int32)
    predt = prediction.T
    corrc = _sc_gather(target, predt)
    out = pl.pallas_call(
        _tc_body,
        grid=(K,),
        in_specs=[
            pl.BlockSpec((1, B), lambda k: (0, 0)),
            pl.BlockSpec((VB, B), lambda k: (k, 0)),
        ],
        out_specs=pl.BlockSpec(memory_space=pltpu.SMEM),
        out_shape=jax.ShapeDtypeStruct((1,), jnp.float32),
        scratch_shapes=[pltpu.SMEM((1,), jnp.float32)],
    )(corrc.reshape(1, B), predt)
    return out
